# baseline (speedup 1.0000x reference)
import functools

import jax
import jax.numpy as jnp
from jax import lax
from jax.experimental import pallas as pl
from jax.experimental.pallas import tpu as pltpu

_PART_FRACS = (0.4, 0.3, 0.3)
_F32_ONE_BITS = 0x3F800000


def _part_bounds(h):
    bounds = []
    start = 0
    for f in _PART_FRACS:
        end = min(start + int(h * f), h)
        bounds.append((start, end))
        start = end
    return bounds


def _pool_body(x_ref, out_ref, *, bounds, w):
    xb = x_ref[0]
    hw = xb.shape[1]
    e = lax.broadcasted_iota(jnp.int32, (4, hw), 1)
    row = e // w
    s = lax.broadcasted_iota(jnp.int32, (4, hw), 0)
    m = (s == 3)
    for i, (lo, hi) in enumerate(bounds):
        m = m | ((s == i) & (row >= lo) & (row < hi))
    mask = m.astype(jnp.float32)
    out_ref[0] = lax.dot_general(
        mask, xb, (((1,), (1,)), ((), ())),
        precision=lax.Precision.HIGHEST, preferred_element_type=jnp.float32)


def _att_body(sums_ref, W1_ref, b1_ref, W2_ref, b2_ref, Wp_ref, bp_ref,
              G1_ref, g1_ref, G2_ref, g2_ref, fa_ref, pw_ref, k_ref,
              *, bounds, h, w):
    c = fa_ref.shape[1]
    sums = sums_ref[...]
    gp = sums[:, 3, :] * (1.0 / (h * w))

    def dot_t(a, b):
        return lax.dot_general(a, b, (((1,), (1,)), ((), ())),
                               precision=lax.Precision.DEFAULT,
                               preferred_element_type=jnp.float32)

    pooled = []
    for i, (lo, hi) in enumerate(bounds):
        pooled.append(sums[:, i, :] * (1.0 / ((hi - lo) * w)))

    atts = []
    for i in range(3):
        hdn = jax.nn.relu(dot_t(pooled[i], W1_ref[i]) + b1_ref[i:i + 1, :])
        atts.append(jax.nn.sigmoid(dot_t(hdn, W2_ref[i]) + b2_ref[i:i + 1, :]))

    logits = dot_t(gp, Wp_ref[...]) + bp_ref[...]
    mx = jnp.max(logits, axis=1, keepdims=True)
    ex = jnp.exp(logits - mx)
    pw = ex / jnp.sum(ex, axis=1, keepdims=True)

    fused = (pw[:, 0:1] * atts[0] + pw[:, 1:2] * atts[1]
             + pw[:, 2:3] * atts[2])

    hg = jax.nn.relu(dot_t(gp, G1_ref[...]) + g1_ref[...])
    sp_logit = jnp.sum(hg * G2_ref[...], axis=1, keepdims=True)
    sp = jax.nn.sigmoid(sp_logit + g2_ref[...])
    k = jnp.clip((sp * c).astype(jnp.int32), 1, c)

    bits = lax.bitcast_convert_type(fused, jnp.int32)

    def count_gt(t):
        return jnp.sum((bits > t).astype(jnp.int32), axis=1, keepdims=True)

    def v_body(_, carry):
        lo, hi = carry
        mid = (lo + hi) // 2
        pred = count_gt(mid) < k
        return jnp.where(pred, lo, mid + 1), jnp.where(pred, mid, hi)

    lo = jnp.zeros_like(k)
    hi = jnp.full_like(k, _F32_ONE_BITS)
    lo, hi = lax.fori_loop(0, 31, v_body, (lo, hi))
    t = lo
    m = k - count_gt(t)
    eq = bits == t
    idx = lax.broadcasted_iota(jnp.int32, bits.shape, 1)

    def j_body(_, carry):
        jlo, jhi = carry
        mid = (jlo + jhi) // 2
        cnt = jnp.sum((eq & (idx <= mid)).astype(jnp.int32), axis=1,
                      keepdims=True)
        pred = cnt >= m
        return jnp.where(pred, jlo, mid + 1), jnp.where(pred, mid, jhi)

    jlo = jnp.zeros_like(k)
    jhi = jnp.full_like(k, c - 1)
    jlo, jhi = lax.fori_loop(0, 12, j_body, (jlo, jhi))

    sel = (bits > t) | (eq & (idx <= jlo))
    fa_ref[...] = jnp.where(sel, fused, 0.0)
    pw_ref[...] = pw
    k_ref[...] = k


def _apply_body(x_ref, fa_ref, out1_ref, out2_ref):
    xb = x_ref[0]
    f = fa_ref[0]
    o1 = xb * f
    out1_ref[0] = o1
    out2_ref[0] = xb - o1


def kernel(x, W1, b1, W2, b2, Wp, bp, G1, g1, G2, g2):
    b, c, h, w = x.shape
    hw = h * w
    bounds = _part_bounds(h)
    x3 = x.reshape(b, c, hw)

    CB = 256
    n_cb = c // CB

    sums = pl.pallas_call(
        functools.partial(_pool_body, bounds=bounds, w=w),
        grid=(b, n_cb),
        in_specs=[pl.BlockSpec((1, CB, hw), lambda i, j: (i, j, 0))],
        out_specs=pl.BlockSpec((1, 4, CB), lambda i, j: (i, 0, j)),
        out_shape=jax.ShapeDtypeStruct((b, 4, c), jnp.float32),
    )(x3)

    fa, pw, _ = pl.pallas_call(
        functools.partial(_att_body, bounds=bounds, h=h, w=w),
        out_shape=(
            jax.ShapeDtypeStruct((b, c), jnp.float32),
            jax.ShapeDtypeStruct((b, 3), jnp.float32),
            jax.ShapeDtypeStruct((b, 1), jnp.int32),
        ),
    )(sums, W1, b1, W2, b2, Wp, bp.reshape(1, 3), G1, g1.reshape(1, -1),
      G2, jnp.broadcast_to(g2.reshape(1, 1), (b, 1)))

    out1, out2 = pl.pallas_call(
        _apply_body,
        grid=(b, n_cb),
        in_specs=[
            pl.BlockSpec((1, CB, hw), lambda i, j: (i, j, 0)),
            pl.BlockSpec((1, CB, 1), lambda i, j: (i, j, 0)),
        ],
        out_specs=[
            pl.BlockSpec((1, CB, hw), lambda i, j: (i, j, 0)),
            pl.BlockSpec((1, CB, hw), lambda i, j: (i, j, 0)),
        ],
        out_shape=(
            jax.ShapeDtypeStruct((b, c, hw), jnp.float32),
            jax.ShapeDtypeStruct((b, c, hw), jnp.float32),
        ),
    )(x3, fa.reshape(b, c, 1))

    return (out1.reshape(b, c, h, w), out2.reshape(b, c, h, w),
            pw.reshape(b, 3, 1, 1))


# speedup vs baseline: 1.0283x; 1.0283x over previous
import functools

import jax
import jax.numpy as jnp
from jax import lax
from jax.experimental import pallas as pl
from jax.experimental.pallas import tpu as pltpu

_PART_FRACS = (0.4, 0.3, 0.3)
_F32_ONE_BITS = 0x3F800000


def _part_bounds(h):
    bounds = []
    start = 0
    for f in _PART_FRACS:
        end = min(start + int(h * f), h)
        bounds.append((start, end))
        start = end
    return bounds


def _pool_body(x_ref, out_ref, *, bounds, w):
    xb = x_ref[...]
    hw = xb.shape[1]
    e = lax.broadcasted_iota(jnp.int32, (4, hw), 1)
    row = e // w
    s = lax.broadcasted_iota(jnp.int32, (4, hw), 0)
    m = (s == 3)
    for i, (lo, hi) in enumerate(bounds):
        m = m | ((s == i) & (row >= lo) & (row < hi))
    mask = m.astype(jnp.float32)
    out_ref[...] = lax.dot_general(
        mask, xb, (((1,), (1,)), ((), ())),
        precision=lax.Precision.HIGHEST, preferred_element_type=jnp.float32)


def _att_body(sums_ref, W1_ref, b1_ref, W2_ref, b2_ref, Wp_ref, bp_ref,
              G1_ref, g1_ref, G2_ref, g2_ref, fa_ref, pw_ref, k_ref,
              *, bounds, h, w):
    c = fa_ref.shape[1]
    gp = sums_ref[3] * (1.0 / (h * w))

    def dot_t(a, b):
        return lax.dot_general(a, b, (((1,), (1,)), ((), ())),
                               precision=lax.Precision.DEFAULT,
                               preferred_element_type=jnp.float32)

    atts = []
    for i, (lo, hi) in enumerate(bounds):
        pooled = sums_ref[i] * (1.0 / ((hi - lo) * w))
        hdn = jax.nn.relu(dot_t(pooled, W1_ref[i]) + b1_ref[i:i + 1, :])
        atts.append(jax.nn.sigmoid(dot_t(hdn, W2_ref[i]) + b2_ref[i:i + 1, :]))

    logits = dot_t(gp, Wp_ref[...]) + bp_ref[...]
    mx = jnp.max(logits, axis=1, keepdims=True)
    ex = jnp.exp(logits - mx)
    pw = ex / jnp.sum(ex, axis=1, keepdims=True)

    fused = (pw[:, 0:1] * atts[0] + pw[:, 1:2] * atts[1]
             + pw[:, 2:3] * atts[2])

    hg = jax.nn.relu(dot_t(gp, G1_ref[...]) + g1_ref[...])
    sp_logit = jnp.sum(hg * G2_ref[...], axis=1, keepdims=True)
    sp = jax.nn.sigmoid(sp_logit + g2_ref[...])
    k = jnp.clip((sp * c).astype(jnp.int32), 1, c)

    bits = lax.bitcast_convert_type(fused, jnp.int32)

    def count_gt(t):
        return jnp.sum((bits > t).astype(jnp.int32), axis=1, keepdims=True)

    def v_body(_, carry):
        lo, hi = carry
        mid = (lo + hi) // 2
        pred = count_gt(mid) < k
        return jnp.where(pred, lo, mid + 1), jnp.where(pred, mid, hi)

    lo = jnp.zeros_like(k)
    hi = jnp.full_like(k, _F32_ONE_BITS)
    lo, hi = lax.fori_loop(0, 31, v_body, (lo, hi))
    t = lo
    m = k - count_gt(t)
    eq = bits == t
    idx = lax.broadcasted_iota(jnp.int32, bits.shape, 1)

    def j_body(_, carry):
        jlo, jhi = carry
        mid = (jlo + jhi) // 2
        cnt = jnp.sum((eq & (idx <= mid)).astype(jnp.int32), axis=1,
                      keepdims=True)
        pred = cnt >= m
        return jnp.where(pred, jlo, mid + 1), jnp.where(pred, mid, jhi)

    jlo = jnp.zeros_like(k)
    jhi = jnp.full_like(k, c - 1)
    jlo, jhi = lax.fori_loop(0, 12, j_body, (jlo, jhi))

    sel = (bits > t) | (eq & (idx <= jlo))
    fa_ref[...] = jnp.where(sel, fused, 0.0)
    pw_ref[...] = pw
    k_ref[...] = k


def _apply_body(x_ref, fa_ref, out1_ref, out2_ref):
    xb = x_ref[...]
    f = fa_ref[...]
    o1 = xb * f
    out1_ref[...] = o1
    out2_ref[...] = xb - o1


def kernel(x, W1, b1, W2, b2, Wp, bp, G1, g1, G2, g2):
    b, c, h, w = x.shape
    hw = h * w
    bounds = _part_bounds(h)
    xr = x.reshape(b * c, hw)

    RB = 4096
    n_rb = (b * c) // RB

    sums = pl.pallas_call(
        functools.partial(_pool_body, bounds=bounds, w=w),
        grid=(n_rb,),
        in_specs=[pl.BlockSpec((RB, hw), lambda i: (i, 0))],
        out_specs=pl.BlockSpec((4, RB), lambda i: (0, i)),
        out_shape=jax.ShapeDtypeStruct((4, b * c), jnp.float32),
        compiler_params=pltpu.CompilerParams(
            dimension_semantics=("arbitrary",)),
    )(xr)

    fa, pw, _ = pl.pallas_call(
        functools.partial(_att_body, bounds=bounds, h=h, w=w),
        out_shape=(
            jax.ShapeDtypeStruct((b, c), jnp.float32),
            jax.ShapeDtypeStruct((b, 3), jnp.float32),
            jax.ShapeDtypeStruct((b, 1), jnp.int32),
        ),
    )(sums.reshape(4, b, c), W1, b1, W2, b2, Wp, bp.reshape(1, 3), G1,
      g1.reshape(1, -1), G2, jnp.broadcast_to(g2.reshape(1, 1), (b, 1)))

    out1, out2 = pl.pallas_call(
        _apply_body,
        grid=(n_rb,),
        in_specs=[
            pl.BlockSpec((RB, hw), lambda i: (i, 0)),
            pl.BlockSpec((RB, 1), lambda i: (i, 0)),
        ],
        out_specs=[
            pl.BlockSpec((RB, hw), lambda i: (i, 0)),
            pl.BlockSpec((RB, hw), lambda i: (i, 0)),
        ],
        out_shape=(
            jax.ShapeDtypeStruct((b * c, hw), jnp.float32),
            jax.ShapeDtypeStruct((b * c, hw), jnp.float32),
        ),
        compiler_params=pltpu.CompilerParams(
            dimension_semantics=("arbitrary",)),
    )(xr, fa.reshape(b * c, 1))

    return (out1.reshape(b, c, h, w), out2.reshape(b, c, h, w),
            pw.reshape(b, 3, 1, 1))


# speedup vs baseline: 1.2376x; 1.2035x over previous
import functools

import jax
import jax.numpy as jnp
from jax import lax
from jax.experimental import pallas as pl
from jax.experimental.pallas import tpu as pltpu

_PART_FRACS = (0.4, 0.3, 0.3)
_F32_ONE_BITS = 0x3F800000


def _part_bounds(h):
    bounds = []
    start = 0
    for f in _PART_FRACS:
        end = min(start + int(h * f), h)
        bounds.append((start, end))
        start = end
    return bounds


def _pool_body(x_ref, out_ref, *, bounds, w, hw):
    xb = x_ref[0]
    lanes = xb.shape[1]
    l = lax.broadcasted_iota(jnp.int32, (8, lanes), 1)
    o = l % hw
    q = l // hw
    edges = [hi * w for (_, hi) in bounds]
    pidx = ((o >= edges[0]).astype(jnp.int32)
            + (o >= edges[1]).astype(jnp.int32)
            + (o >= edges[2]).astype(jnp.int32))
    s = lax.broadcasted_iota(jnp.int32, (8, lanes), 0)
    mask = (s == q * 4 + pidx).astype(jnp.float32)
    out_ref[0] = lax.dot_general(
        mask, xb, (((1,), (1,)), ((), ())),
        precision=lax.Precision.HIGHEST, preferred_element_type=jnp.float32)


def _att_body(sums_ref, W1_ref, b1_ref, W2_ref, b2_ref, Wp_ref, bp_ref,
              G1_ref, g1_ref, G2_ref, g2_ref, fa_ref, pw_ref,
              *, bounds, h, w):
    c = fa_ref.shape[1]
    gp = ((sums_ref[0] + sums_ref[1] + sums_ref[2] + sums_ref[3])
          * (1.0 / (h * w)))

    def dot_t(a, b):
        return lax.dot_general(a, b, (((1,), (1,)), ((), ())),
                               precision=lax.Precision.DEFAULT,
                               preferred_element_type=jnp.float32)

    atts = []
    for i, (lo, hi) in enumerate(bounds):
        pooled = sums_ref[i] * (1.0 / ((hi - lo) * w))
        hdn = jax.nn.relu(dot_t(pooled, W1_ref[i]) + b1_ref[i:i + 1, :])
        atts.append(jax.nn.sigmoid(dot_t(hdn, W2_ref[i]) + b2_ref[i:i + 1, :]))

    logits = dot_t(gp, Wp_ref[...]) + bp_ref[...]
    mx = jnp.max(logits, axis=1, keepdims=True)
    ex = jnp.exp(logits - mx)
    pw = ex / jnp.sum(ex, axis=1, keepdims=True)

    fused = (pw[:, 0:1] * atts[0] + pw[:, 1:2] * atts[1]
             + pw[:, 2:3] * atts[2])

    hg = jax.nn.relu(dot_t(gp, G1_ref[...]) + g1_ref[...])
    sp_logit = jnp.sum(hg * G2_ref[...], axis=1, keepdims=True)
    sp = jax.nn.sigmoid(sp_logit + g2_ref[...])
    k = jnp.clip((sp * c).astype(jnp.int32), 1, c)

    bits = lax.bitcast_convert_type(fused, jnp.int32)

    def count_gt(t):
        return jnp.sum((bits > t).astype(jnp.int32), axis=1, keepdims=True)

    def v_body(_, carry):
        lo, hi = carry
        mid = (lo + hi) // 2
        pred = count_gt(mid) < k
        return jnp.where(pred, lo, mid + 1), jnp.where(pred, mid, hi)

    lo = jnp.zeros_like(k)
    hi = jnp.full_like(k, _F32_ONE_BITS)
    lo, hi = lax.fori_loop(0, 31, v_body, (lo, hi))
    t = lo
    m = k - count_gt(t)
    eq = bits == t
    idx = lax.broadcasted_iota(jnp.int32, bits.shape, 1)

    def j_body(_, carry):
        jlo, jhi = carry
        mid = (jlo + jhi) // 2
        cnt = jnp.sum((eq & (idx <= mid)).astype(jnp.int32), axis=1,
                      keepdims=True)
        pred = cnt >= m
        return jnp.where(pred, jlo, mid + 1), jnp.where(pred, mid, jhi)

    jlo = jnp.zeros_like(k)
    jhi = jnp.full_like(k, c - 1)
    jlo, jhi = lax.fori_loop(0, 12, j_body, (jlo, jhi))

    sel = (bits > t) | (eq & (idx <= jlo))
    fa_ref[...] = jnp.where(sel, fused, 0.0)
    pw_ref[...] = pw


def _apply_body(x_ref, faeT_ref, faoT_ref, out1_ref, out2_ref, *, hw):
    i = pl.program_id(0)
    xb = x_ref[0]
    fe = faeT_ref[...]
    fo = faoT_ref[...]
    oh = (lax.broadcasted_iota(jnp.int32, fe.shape, 1) == i).astype(
        jnp.float32)
    le = jnp.sum(fe * oh, axis=1, keepdims=True)
    lo = jnp.sum(fo * oh, axis=1, keepdims=True)
    lane = lax.broadcasted_iota(jnp.int32, xb.shape, 1)
    faexp = jnp.where(lane < hw, le, lo)
    o1 = xb * faexp
    out1_ref[0] = o1
    out2_ref[0] = xb - o1


def kernel(x, W1, b1, W2, b2, Wp, bp, G1, g1, G2, g2):
    b, c, h, w = x.shape
    hw = h * w
    G = c // 2
    bounds = _part_bounds(h)
    x4 = x.reshape(b, G, 2 * hw)

    sums8 = pl.pallas_call(
        functools.partial(_pool_body, bounds=bounds, w=w, hw=hw),
        grid=(b,),
        in_specs=[pl.BlockSpec((1, G, 2 * hw), lambda i: (i, 0, 0))],
        out_specs=pl.BlockSpec((1, 8, G), lambda i: (i, 0, 0)),
        out_shape=jax.ShapeDtypeStruct((b, 8, G), jnp.float32),
    )(x4)

    sums4 = sums8.reshape(b, 2, 4, G).transpose(2, 0, 3, 1).reshape(4, b, c)

    fa, pw = pl.pallas_call(
        functools.partial(_att_body, bounds=bounds, h=h, w=w),
        out_shape=(
            jax.ShapeDtypeStruct((b, c), jnp.float32),
            jax.ShapeDtypeStruct((b, 3), jnp.float32),
        ),
    )(sums4, W1, b1, W2, b2, Wp, bp.reshape(1, 3), G1,
      g1.reshape(1, -1), G2, jnp.broadcast_to(g2.reshape(1, 1), (b, 1)))

    faeT = fa[:, 0::2].T
    faoT = fa[:, 1::2].T

    out1, out2 = pl.pallas_call(
        functools.partial(_apply_body, hw=hw),
        grid=(b,),
        in_specs=[
            pl.BlockSpec((1, G, 2 * hw), lambda i: (i, 0, 0)),
            pl.BlockSpec((G, b), lambda i: (0, 0)),
            pl.BlockSpec((G, b), lambda i: (0, 0)),
        ],
        out_specs=[
            pl.BlockSpec((1, G, 2 * hw), lambda i: (i, 0, 0)),
            pl.BlockSpec((1, G, 2 * hw), lambda i: (i, 0, 0)),
        ],
        out_shape=(
            jax.ShapeDtypeStruct((b, G, 2 * hw), jnp.float32),
            jax.ShapeDtypeStruct((b, G, 2 * hw), jnp.float32),
        ),
    )(x4, faeT, faoT)

    return (out1.reshape(b, c, h, w), out2.reshape(b, c, h, w),
            pw.reshape(b, 3, 1, 1))


# speedup vs baseline: 5.4365x; 4.3926x over previous
import functools

import jax
import jax.numpy as jnp
from jax import lax
from jax.experimental import pallas as pl
from jax.experimental.pallas import tpu as pltpu

_PART_FRACS = (0.4, 0.3, 0.3)
_F32_ONE_BITS = 0x3F800000


def _part_bounds(h):
    bounds = []
    start = 0
    for f in _PART_FRACS:
        end = min(start + int(h * f), h)
        bounds.append((start, end))
        start = end
    return bounds


def _pool_body(x_ref, out_ref, *, bounds, w, hw):
    xb = x_ref[0]
    e = lax.broadcasted_iota(jnp.int32, (4, hw), 1)
    row = e // w
    s = lax.broadcasted_iota(jnp.int32, (4, hw), 0)
    m = (s == 3) & (row >= bounds[2][1])
    for i, (lo, hi) in enumerate(bounds):
        m = m | ((s == i) & (row >= lo) & (row < hi))
    mask = m.astype(jnp.float32)
    out_ref[0] = lax.dot_general(
        mask, xb, (((1,), (0,)), ((), ())),
        precision=lax.Precision.HIGHEST, preferred_element_type=jnp.float32)


def _att_body(sums_ref, W1_ref, b1_ref, W2_ref, b2_ref, Wp_ref, bp_ref,
              G1_ref, g1_ref, G2_ref, g2_ref, fa_ref, pw_ref,
              *, bounds, h, w):
    c = fa_ref.shape[1]
    sums = sums_ref[...]
    gp = ((sums[:, 0, :] + sums[:, 1, :] + sums[:, 2, :] + sums[:, 3, :])
          * (1.0 / (h * w)))

    def dot_t(a, b):
        return lax.dot_general(a, b, (((1,), (1,)), ((), ())),
                               precision=lax.Precision.DEFAULT,
                               preferred_element_type=jnp.float32)

    atts = []
    for i, (lo, hi) in enumerate(bounds):
        pooled = sums[:, i, :] * (1.0 / ((hi - lo) * w))
        hdn = jax.nn.relu(dot_t(pooled, W1_ref[i]) + b1_ref[i:i + 1, :])
        atts.append(jax.nn.sigmoid(dot_t(hdn, W2_ref[i]) + b2_ref[i:i + 1, :]))

    logits = dot_t(gp, Wp_ref[...]) + bp_ref[...]
    mx = jnp.max(logits, axis=1, keepdims=True)
    ex = jnp.exp(logits - mx)
    pw = ex / jnp.sum(ex, axis=1, keepdims=True)

    fused = (pw[:, 0:1] * atts[0] + pw[:, 1:2] * atts[1]
             + pw[:, 2:3] * atts[2])

    hg = jax.nn.relu(dot_t(gp, G1_ref[...]) + g1_ref[...])
    sp_logit = jnp.sum(hg * G2_ref[...], axis=1, keepdims=True)
    sp = jax.nn.sigmoid(sp_logit + g2_ref[...])
    k = jnp.clip((sp * c).astype(jnp.int32), 1, c)

    bits = lax.bitcast_convert_type(fused, jnp.int32)

    def count_gt(t):
        return jnp.sum((bits > t).astype(jnp.int32), axis=1, keepdims=True)

    def v_body(_, carry):
        lo, hi = carry
        mid = (lo + hi) // 2
        pred = count_gt(mid) < k
        return jnp.where(pred, lo, mid + 1), jnp.where(pred, mid, hi)

    lo = jnp.zeros_like(k)
    hi = jnp.full_like(k, _F32_ONE_BITS)
    lo, hi = lax.fori_loop(0, 31, v_body, (lo, hi))
    t = lo
    m = k - count_gt(t)
    eq = bits == t
    idx = lax.broadcasted_iota(jnp.int32, bits.shape, 1)

    def j_body(_, carry):
        jlo, jhi = carry
        mid = (jlo + jhi) // 2
        cnt = jnp.sum((eq & (idx <= mid)).astype(jnp.int32), axis=1,
                      keepdims=True)
        pred = cnt >= m
        return jnp.where(pred, jlo, mid + 1), jnp.where(pred, mid, jhi)

    jlo = jnp.zeros_like(k)
    jhi = jnp.full_like(k, c - 1)
    jlo, jhi = lax.fori_loop(0, 12, j_body, (jlo, jhi))

    sel = (bits > t) | (eq & (idx <= jlo))
    fa_ref[...] = jnp.where(sel, fused, 0.0)
    pw_ref[...] = pw


def _apply_body(x_ref, fa_ref, out1_ref, out2_ref):
    xb = x_ref[0]
    f = fa_ref[0]
    o1 = xb * f
    out1_ref[0] = o1
    out2_ref[0] = xb - o1


def kernel(x, W1, b1, W2, b2, Wp, bp, G1, g1, G2, g2):
    b, c, h, w = x.shape
    hw = h * w
    bounds = _part_bounds(h)
    xcl = x.transpose(0, 2, 3, 1).reshape(b, hw, c)

    sums = pl.pallas_call(
        functools.partial(_pool_body, bounds=bounds, w=w, hw=hw),
        grid=(b,),
        in_specs=[pl.BlockSpec((1, hw, c), lambda i: (i, 0, 0))],
        out_specs=pl.BlockSpec((1, 4, c), lambda i: (i, 0, 0)),
        out_shape=jax.ShapeDtypeStruct((b, 4, c), jnp.float32),
    )(xcl)

    fa, pw = pl.pallas_call(
        functools.partial(_att_body, bounds=bounds, h=h, w=w),
        out_shape=(
            jax.ShapeDtypeStruct((b, c), jnp.float32),
            jax.ShapeDtypeStruct((b, 3), jnp.float32),
        ),
    )(sums, W1, b1, W2, b2, Wp, bp.reshape(1, 3), G1,
      g1.reshape(1, -1), G2, jnp.broadcast_to(g2.reshape(1, 1), (b, 1)))

    out1, out2 = pl.pallas_call(
        _apply_body,
        grid=(b,),
        in_specs=[
            pl.BlockSpec((1, hw, c), lambda i: (i, 0, 0)),
            pl.BlockSpec((1, 1, c), lambda i: (i, 0, 0)),
        ],
        out_specs=[
            pl.BlockSpec((1, hw, c), lambda i: (i, 0, 0)),
            pl.BlockSpec((1, hw, c), lambda i: (i, 0, 0)),
        ],
        out_shape=(
            jax.ShapeDtypeStruct((b, hw, c), jnp.float32),
            jax.ShapeDtypeStruct((b, hw, c), jnp.float32),
        ),
    )(xcl, fa.reshape(b, 1, c))

    def back(o):
        return o.reshape(b, h, w, c).transpose(0, 3, 1, 2)

    return back(out1), back(out2), pw.reshape(b, 3, 1, 1)
